# baseline (speedup 1.0000x reference)
import jax
import jax.numpy as jnp
import numpy as np
from jax.experimental import pallas as pl
from jax.experimental.pallas import tpu as pltpu
from jax.experimental.pallas import tpu_sc as plsc

_PERCENTAGE = 0.5
_SEED = 0
_B = 16

def _subset_rows():
    try:
        cpu = jax.devices("cpu")[0]
        with jax.default_device(cpu):
            perm = np.asarray(jax.random.permutation(jax.random.key(_SEED), _B))
        sel = sorted(int(v) for v in perm[: int(_B * _PERCENTAGE)])
    except Exception:
        sel = [0, 1, 4, 5, 6, 8, 12, 13]
    unsel = sorted(set(range(_B)) - set(sel))
    return tuple(sel), tuple(unsel)

_SEL, _UNSEL = _subset_rows()

_R = 96
_C = 224 * 224
_F = _R * _C
_N_CHUNKS = 4
_CHUNK = _F // _N_CHUNKS

_ROWS_PER_BLOCK = 16


def _sc_copy_rows(x):
    mesh = plsc.VectorSubcoreMesh(core_axis_name="c", subcore_axis_name="s")

    @pl.kernel(
        out_type=jax.ShapeDtypeStruct((_B, _F), jnp.float32),
        mesh=mesh,
        scratch_types=[pltpu.SemaphoreType.DMA],
    )
    def sc_copy(x_hbm, o_hbm, sem):
        c = jax.lax.axis_index("c")
        s = jax.lax.axis_index("s")
        flat = c * 16 + s
        for u in range(len(_UNSEL) * _N_CHUNKS):
            row = _UNSEL[u // _N_CHUNKS]
            col0 = (u % _N_CHUNKS) * _CHUNK

            @pl.when(flat == u)
            def _():
                pltpu.async_copy(
                    x_hbm.at[row, pl.ds(col0, _CHUNK)],
                    o_hbm.at[row, pl.ds(col0, _CHUNK)],
                    sem,
                ).wait()

    return sc_copy(x)


def _tc_relu_body(sel_ref, x_ref, tmp_hbm, o_ref):
    o_ref[...] = jnp.maximum(x_ref[...], 0.0)


def _tc_relu_selected(xv, tmp):
    sel_arr = jnp.asarray(_SEL, dtype=jnp.int32)
    grid = (len(_SEL), _R // _ROWS_PER_BLOCK)
    return pl.pallas_call(
        _tc_relu_body,
        grid_spec=pltpu.PrefetchScalarGridSpec(
            num_scalar_prefetch=1,
            grid=grid,
            in_specs=[
                pl.BlockSpec((1, _ROWS_PER_BLOCK, _C),
                             lambda i, r, sel: (sel[i], r, 0)),
                pl.BlockSpec(memory_space=pl.ANY),
            ],
            out_specs=pl.BlockSpec((1, _ROWS_PER_BLOCK, _C),
                                   lambda i, r, sel: (sel[i], r, 0)),
        ),
        out_shape=jax.ShapeDtypeStruct((_B, _R, _C), jnp.float32),
        input_output_aliases={2: 0},
    )(sel_arr, xv, tmp)


def kernel(x):
    xf = x.reshape(_B, _F)
    tmp = _sc_copy_rows(xf)
    out = _tc_relu_selected(x.reshape(_B, _R, _C), tmp.reshape(_B, _R, _C))
    return out.reshape(x.shape)


# speedup vs baseline: 15.8681x; 15.8681x over previous
import jax
import jax.numpy as jnp
from jax.experimental import pallas as pl
from jax.experimental.pallas import tpu as pltpu

_PERCENTAGE = 0.5
_SEED = 0

_ROWS_PER_BLOCK = 32


def _masked_relu_body(mask_ref, x_ref, o_ref):
    b = pl.program_id(0)
    sel = mask_ref[b] != 0

    @pl.when(sel)
    def _():
        o_ref[...] = jnp.maximum(x_ref[...], 0.0)

    @pl.when(jnp.logical_not(sel))
    def _():
        o_ref[...] = x_ref[...]


def kernel(x):
    B = x.shape[0]
    subset_size = int(B * _PERCENTAGE)
    perm = jax.random.permutation(jax.random.key(_SEED), B)
    idx = perm[:subset_size]
    mask = jnp.zeros((B,), jnp.int32).at[idx].set(1)

    R = x.shape[1]
    C = x.shape[2] * x.shape[3]
    xv = x.reshape(B, R, C)

    grid = (B, R // _ROWS_PER_BLOCK)
    out = pl.pallas_call(
        _masked_relu_body,
        grid_spec=pltpu.PrefetchScalarGridSpec(
            num_scalar_prefetch=1,
            grid=grid,
            in_specs=[
                pl.BlockSpec((1, _ROWS_PER_BLOCK, C), lambda b, r, m: (b, r, 0)),
            ],
            out_specs=pl.BlockSpec((1, _ROWS_PER_BLOCK, C), lambda b, r, m: (b, r, 0)),
        ),
        out_shape=jax.ShapeDtypeStruct((B, R, C), x.dtype),
    )(mask, xv)
    return out.reshape(x.shape)
